# baseline (speedup 1.0000x reference)
import functools

import jax
import jax.numpy as jnp
from jax.experimental import pallas as pl
from jax.experimental.pallas import tpu as pltpu

N_NODES = 10000
HID = 256
N_LAYERS = 2
ROW_BLK = 2000
N_ROW_BLKS = N_NODES // ROW_BLK
EPS = 1e-5


def _msg_kernel(h_ref, w_ref, b_ref, m_ref):
    h = h_ref[0]
    m_ref[0] = jnp.dot(h, w_ref[0], preferred_element_type=jnp.float32) + b_ref[0]


def _msg_bn_kernel(x_ref, stats_ref, gamma_ref, beta_ref, w_ref, b_ref,
                   h_ref, m_ref):
    x = x_ref[0]
    mu = stats_ref[0, 0:1, :] * (1.0 / N_NODES)
    ex2 = stats_ref[0, 1:2, :] * (1.0 / N_NODES)
    var = ex2 - mu * mu
    rstd = jax.lax.rsqrt(var + EPS)
    h = jnp.tanh((x - mu) * rstd * gamma_ref[0] + beta_ref[0])
    h_ref[0] = h
    m_ref[0] = jnp.dot(h, w_ref[0], preferred_element_type=jnp.float32) + b_ref[0]


def _msg_call(h2, W2, b2):
    return pl.pallas_call(
        _msg_kernel,
        grid=(2, N_ROW_BLKS),
        in_specs=[
            pl.BlockSpec((1, ROW_BLK, HID), lambda t, i: (t, i, 0)),
            pl.BlockSpec((1, HID, HID), lambda t, i: (t, 0, 0)),
            pl.BlockSpec((1, 1, HID), lambda t, i: (t, 0, 0)),
        ],
        out_specs=pl.BlockSpec((1, ROW_BLK, HID), lambda t, i: (t, i, 0)),
        out_shape=jax.ShapeDtypeStruct((2, N_NODES, HID), jnp.float32),
    )(h2, W2, b2)


def _msg_bn_call(x2, stats2, gamma2, beta2, W2, b2):
    return pl.pallas_call(
        _msg_bn_kernel,
        grid=(2, N_ROW_BLKS),
        in_specs=[
            pl.BlockSpec((1, ROW_BLK, HID), lambda t, i: (t, i, 0)),
            pl.BlockSpec((1, 2, HID), lambda t, i: (t, 0, 0)),
            pl.BlockSpec((1, 1, HID), lambda t, i: (t, 0, 0)),
            pl.BlockSpec((1, 1, HID), lambda t, i: (t, 0, 0)),
            pl.BlockSpec((1, HID, HID), lambda t, i: (t, 0, 0)),
            pl.BlockSpec((1, 1, HID), lambda t, i: (t, 0, 0)),
        ],
        out_specs=[
            pl.BlockSpec((1, ROW_BLK, HID), lambda t, i: (t, i, 0)),
            pl.BlockSpec((1, ROW_BLK, HID), lambda t, i: (t, i, 0)),
        ],
        out_shape=[
            jax.ShapeDtypeStruct((2, N_NODES, HID), jnp.float32),
            jax.ShapeDtypeStruct((2, N_NODES, HID), jnp.float32),
        ],
    )(x2, stats2, gamma2, beta2, W2, b2)


def _gru_kernel(a_ref, h_ref, wi_ref, wh_ref, bi_ref, bh_ref,
                hp_ref, stats_ref):
    i = pl.program_id(1)
    a = a_ref[0]
    h = h_ref[0]
    gi = jnp.dot(a, wi_ref[0], preferred_element_type=jnp.float32) + bi_ref[0]
    gh = jnp.dot(h, wh_ref[0], preferred_element_type=jnp.float32) + bh_ref[0]
    r = jax.nn.sigmoid(gi[:, :HID] + gh[:, :HID])
    z = jax.nn.sigmoid(gi[:, HID:2 * HID] + gh[:, HID:2 * HID])
    n = jnp.tanh(gi[:, 2 * HID:] + r * gh[:, 2 * HID:])
    hp = (1.0 - z) * n + z * h
    hp_ref[0] = hp
    s = jnp.sum(hp, axis=0, keepdims=True)
    s2 = jnp.sum(hp * hp, axis=0, keepdims=True)
    blk = jnp.concatenate([s, s2], axis=0)

    @pl.when(i == 0)
    def _():
        stats_ref[0] = blk

    @pl.when(i > 0)
    def _():
        stats_ref[0] += blk


def _gru_call(a2, h2, Wi2, Wh2, bi2, bh2):
    return pl.pallas_call(
        _gru_kernel,
        grid=(2, N_ROW_BLKS),
        in_specs=[
            pl.BlockSpec((1, ROW_BLK, HID), lambda t, i: (t, i, 0)),
            pl.BlockSpec((1, ROW_BLK, HID), lambda t, i: (t, i, 0)),
            pl.BlockSpec((1, HID, 3 * HID), lambda t, i: (t, 0, 0)),
            pl.BlockSpec((1, HID, 3 * HID), lambda t, i: (t, 0, 0)),
            pl.BlockSpec((1, 1, 3 * HID), lambda t, i: (t, 0, 0)),
            pl.BlockSpec((1, 1, 3 * HID), lambda t, i: (t, 0, 0)),
        ],
        out_specs=[
            pl.BlockSpec((1, ROW_BLK, HID), lambda t, i: (t, i, 0)),
            pl.BlockSpec((1, 2, HID), lambda t, i: (t, 0, 0)),
        ],
        out_shape=[
            jax.ShapeDtypeStruct((2, N_NODES, HID), jnp.float32),
            jax.ShapeDtypeStruct((2, 2, HID), jnp.float32),
        ],
    )(a2, h2, Wi2, Wh2, bi2, bh2)


def _bn_kernel(x_ref, stats_ref, gamma_ref, beta_ref, h_ref):
    x = x_ref[0]
    mu = stats_ref[0, 0:1, :] * (1.0 / N_NODES)
    ex2 = stats_ref[0, 1:2, :] * (1.0 / N_NODES)
    var = ex2 - mu * mu
    rstd = jax.lax.rsqrt(var + EPS)
    h_ref[0] = jnp.tanh((x - mu) * rstd * gamma_ref[0] + beta_ref[0])


def _bn_call(x2, stats2, gamma2, beta2):
    return pl.pallas_call(
        _bn_kernel,
        grid=(2, N_ROW_BLKS),
        in_specs=[
            pl.BlockSpec((1, ROW_BLK, HID), lambda t, i: (t, i, 0)),
            pl.BlockSpec((1, 2, HID), lambda t, i: (t, 0, 0)),
            pl.BlockSpec((1, 1, HID), lambda t, i: (t, 0, 0)),
            pl.BlockSpec((1, 1, HID), lambda t, i: (t, 0, 0)),
        ],
        out_specs=pl.BlockSpec((1, ROW_BLK, HID), lambda t, i: (t, i, 0)),
        out_shape=jax.ShapeDtypeStruct((2, N_NODES, HID), jnp.float32),
    )(x2, stats2, gamma2, beta2)


COL_BLK = 2048
N_COL_BLKS = -(-N_NODES // COL_BLK)


def _score_kernel(qr_ref, qi_ref, hr_ref, hi_ref, o_ref):
    d = (((1,), (1,)), ((), ()))
    s = jax.lax.dot_general(qr_ref[...], hr_ref[...], d,
                            preferred_element_type=jnp.float32)
    s += jax.lax.dot_general(qi_ref[...], hi_ref[...], d,
                             preferred_element_type=jnp.float32)
    o_ref[...] = jax.nn.sigmoid(s)


def _score_call(qr, qi, hr, hi, batch):
    return pl.pallas_call(
        _score_kernel,
        grid=(N_COL_BLKS,),
        in_specs=[
            pl.BlockSpec((batch, HID), lambda j: (0, 0)),
            pl.BlockSpec((batch, HID), lambda j: (0, 0)),
            pl.BlockSpec((COL_BLK, HID), lambda j: (j, 0)),
            pl.BlockSpec((COL_BLK, HID), lambda j: (j, 0)),
        ],
        out_specs=pl.BlockSpec((batch, COL_BLK), lambda j: (0, j)),
        out_shape=jax.ShapeDtypeStruct((batch, N_NODES), jnp.float32),
    )(qr, qi, hr, hi)


@jax.jit
def kernel(graph_nodes_idx, edge_index, e1, rel, word_emb, rel_real, rel_img,
           Wmsg_r, bmsg_r, Wi_r, Wh_r, bi_r, bh_r, gamma_r, beta_r,
           Wmsg_i, bmsg_i, Wi_i, Wh_i, bi_i, bh_i, gamma_i, beta_i):
    node_feat = jnp.mean(word_emb[graph_nodes_idx], axis=1)
    src = edge_index[0]
    dst = edge_index[1]

    def stk(xr, xi):
        return jnp.stack([xr, xi], axis=0)

    Wmsg2 = stk(Wmsg_r, Wmsg_i)
    bmsg2 = stk(bmsg_r, bmsg_i)[:, :, None, :]
    Wi2 = stk(Wi_r, Wi_i)
    Wh2 = stk(Wh_r, Wh_i)
    bi2 = stk(bi_r, bi_i)[:, :, None, :]
    bh2 = stk(bh_r, bh_i)[:, :, None, :]
    gamma2 = stk(gamma_r, gamma_i)[:, :, None, :]
    beta2 = stk(beta_r, beta_i)[:, :, None, :]

    h2 = jnp.stack([node_feat, node_feat], axis=0)

    hp2 = None
    stats2 = None
    for l in range(N_LAYERS):
        if l == 0:
            m2 = _msg_call(h2, Wmsg2[:, l], bmsg2[:, l])
        else:
            h2, m2 = _msg_bn_call(hp2, stats2, gamma2[:, l - 1], beta2[:, l - 1],
                                  Wmsg2[:, l], bmsg2[:, l])
        a2 = jnp.zeros_like(m2).at[:, dst].add(m2[:, src])
        hp2, stats2 = _gru_call(a2, h2, Wi2[:, l], Wh2[:, l], bi2[:, l], bh2[:, l])

    h2 = _bn_call(hp2, stats2, gamma2[:, N_LAYERS - 1], beta2[:, N_LAYERS - 1])
    h_r = h2[0]
    h_i = h2[1]

    er = h_r[e1]
    ei = h_i[e1]
    rr = rel_real[rel]
    ri = rel_img[rel]
    qr = er * rr - ei * ri
    qi = ei * rr + er * ri
    batch = e1.shape[0]
    return _score_call(qr, qi, h_r, h_i, batch)
